# baseline (speedup 1.0000x reference)
import functools

import jax
import jax.numpy as jnp
from jax import lax
from jax.experimental import pallas as pl
from jax.experimental.pallas import tpu as pltpu
from jax.experimental.pallas import tpu_sc as plsc

N = 100000
G = 512
NC = 2
NS = 16
NW = NC * NS
L = 16
CHUNK = 3136
ITERS = CHUNK // L
LAST_BASE = N - CHUNK
LAST_SKIP = (NW - 1) * CHUNK - LAST_BASE

_mesh = plsc.VectorSubcoreMesh(
    core_axis_name="c", subcore_axis_name="s", num_cores=2, num_subcores=NS
)


@functools.partial(
    pl.kernel,
    out_type=jax.ShapeDtypeStruct((NC, G), jnp.float32),
    mesh=_mesh,
    compiler_params=pltpu.CompilerParams(
        needs_layout_passes=False, use_tc_tiling_on_sc=False
    ),
    scratch_types=[
        pltpu.VMEM((CHUNK,), jnp.float32),
        pltpu.VMEM((CHUNK,), jnp.int32),
        pltpu.VMEM((G,), jnp.float32),
        pltpu.VMEM_SHARED((NS, G), jnp.float32),
        pltpu.VMEM((NS, G // NS), jnp.float32),
        pltpu.VMEM((G // NS,), jnp.float32),
    ],
)
def _segsum(
    pos_hbm, batch_hbm, out_hbm, pos_v, batch_v, acc_v, shared, red_v, out_v
):
    c = lax.axis_index("c")
    s = lax.axis_index("s")
    w = c * NS + s

    last = w == NW - 1
    base = jnp.where(last, LAST_BASE, w * CHUNK)
    start16 = jnp.where(last, LAST_SKIP, 0)
    pltpu.sync_copy(pos_hbm.at[pl.ds(base, CHUNK)], pos_v)
    pltpu.sync_copy(batch_hbm.at[pl.ds(base, CHUNK)], batch_v)

    for i in range(G // L):
        acc_v[pl.ds(i * L, L)] = jnp.zeros((L,), jnp.float32)

    lane_base = lax.iota(jnp.int32, L) * ITERS
    U = 4

    def body(j, carry):
        for u in range(U):
            k = lane_base + (j * U + u)
            x = plsc.load_gather(pos_v, [k])
            v = x * x * jnp.float32(0.1)
            b = plsc.load_gather(batch_v, [k])
            plsc.addupdate_scatter(acc_v, [b], v, mask=k >= start16)
        return carry

    lax.fori_loop(0, ITERS // U, body, 0)

    pltpu.sync_copy(acc_v, shared.at[s])
    plsc.subcore_barrier()

    W = G // NS
    pltpu.sync_copy(shared.at[:, pl.ds(s * W, W)], red_v)

    def red_body(r, tots):
        return tuple(
            tots[col] + red_v[r, pl.ds(col * L, L)] for col in range(W // L)
        )

    init = tuple(red_v[0, pl.ds(col * L, L)] for col in range(W // L))
    tots = lax.fori_loop(1, NS, red_body, init)
    for col in range(W // L):
        out_v[pl.ds(col * L, L)] = tots[col]
    pltpu.sync_copy(out_v, out_hbm.at[c, pl.ds(s * W, W)])


def kernel(positions, atomic_numbers, edge_index, unit_shifts, batch):
    del atomic_numbers, edge_index, unit_shifts
    partials = _segsum(positions[:, 0], batch)
    return (partials[0] + partials[1]).reshape(G, 1)


# speedup vs baseline: 1.0319x; 1.0319x over previous
import functools

import jax
import jax.numpy as jnp
from jax import lax
from jax.experimental import pallas as pl
from jax.experimental.pallas import tpu as pltpu
from jax.experimental.pallas import tpu_sc as plsc

N = 100000
G = 512
NC = 2
NS = 16
NW = NC * NS
L = 16
CHUNK = 3136
ITERS = CHUNK // L
LAST_BASE = N - CHUNK
LAST_SKIP = (NW - 1) * CHUNK - LAST_BASE

_mesh = plsc.VectorSubcoreMesh(
    core_axis_name="c", subcore_axis_name="s", num_cores=2, num_subcores=NS
)


@functools.partial(
    pl.kernel,
    out_type=jax.ShapeDtypeStruct((NC, G), jnp.float32),
    mesh=_mesh,
    compiler_params=pltpu.CompilerParams(
        needs_layout_passes=False, use_tc_tiling_on_sc=False
    ),
    scratch_types=[
        pltpu.VMEM((CHUNK,), jnp.float32),
        pltpu.VMEM((CHUNK,), jnp.int32),
        pltpu.VMEM((G,), jnp.float32),
        pltpu.VMEM_SHARED((NS, G), jnp.float32),
        pltpu.VMEM((NS, G // NS), jnp.float32),
        pltpu.VMEM((G // NS,), jnp.float32),
        pltpu.SemaphoreType.DMA,
        pltpu.SemaphoreType.DMA,
    ],
)
def _segsum(
    pos_hbm, batch_hbm, out_hbm, pos_v, batch_v, acc_v, shared, red_v, out_v,
    sem_x, sem_b,
):
    c = lax.axis_index("c")
    s = lax.axis_index("s")
    w = c * NS + s

    last = w == NW - 1
    base = jnp.where(last, LAST_BASE, w * CHUNK)
    start16 = jnp.where(last, LAST_SKIP, 0)
    cp_x = pltpu.async_copy(pos_hbm.at[pl.ds(base, CHUNK)], pos_v, sem_x)
    cp_b = pltpu.async_copy(batch_hbm.at[pl.ds(base, CHUNK)], batch_v, sem_b)

    for i in range(G // L):
        acc_v[pl.ds(i * L, L)] = jnp.zeros((L,), jnp.float32)
    cp_x.wait()
    cp_b.wait()

    lane_base = lax.iota(jnp.int32, L) * ITERS
    U = 4

    def body(j, carry):
        for u in range(U):
            k = lane_base + (j * U + u)
            x = plsc.load_gather(pos_v, [k])
            v = x * x * jnp.float32(0.1)
            b = plsc.load_gather(batch_v, [k])
            plsc.addupdate_scatter(acc_v, [b], v, mask=k >= start16)
        return carry

    lax.fori_loop(0, ITERS // U, body, 0)

    pltpu.sync_copy(acc_v, shared.at[s])
    plsc.subcore_barrier()

    W = G // NS
    pltpu.sync_copy(shared.at[:, pl.ds(s * W, W)], red_v)

    def red_body(r, tots):
        return tuple(
            tots[col] + red_v[r, pl.ds(col * L, L)] for col in range(W // L)
        )

    init = tuple(red_v[0, pl.ds(col * L, L)] for col in range(W // L))
    tots = lax.fori_loop(1, NS, red_body, init)
    for col in range(W // L):
        out_v[pl.ds(col * L, L)] = tots[col]
    pltpu.sync_copy(out_v, out_hbm.at[c, pl.ds(s * W, W)])


def kernel(positions, atomic_numbers, edge_index, unit_shifts, batch):
    del atomic_numbers, edge_index, unit_shifts
    partials = _segsum(positions[:, 0], batch)
    return (partials[0] + partials[1]).reshape(G, 1)
